# baseline (speedup 1.0000x reference)
import functools

import jax
import jax.numpy as jnp
from jax import lax
from jax.experimental import pallas as pl
from jax.experimental.pallas import tpu as pltpu
from jax.experimental.pallas import tpu_sc as plsc

TAU = 8
E = 16
C_IN = 16
D_MODEL = 128
SEQ_LEN = 2048
LOW_RANK_D = 32
EPS = 1e-06
B = 4
N = 128
L_OUT = SEQ_LEN - TAU * (E - 1)
BN = B * N

R_BLK = 8

SC_CORES = 2
SC_SUBCORES = 16
SC_WORKERS = SC_CORES * SC_SUBCORES
ROWS_PER_WORKER = BN // SC_WORKERS
N_CHUNKS = N // 16


R_ROWS = 64


def _proj_fused_body(x_ref, a_ref, cw_ref, b_ref, o_ref):
    c = pl.program_id(1)
    xc = x_ref[:, 0, 0, :]
    s_c = lax.dot_general(xc, a_ref[...], (((1,), (1,)), ((), ())),
                          preferred_element_type=jnp.float32)
    p = jnp.dot(s_c, cw_ref[0], preferred_element_type=jnp.float32)

    @pl.when(c == 0)
    def _():
        o_ref[...] = p

    @pl.when(c > 0)
    def _():
        o_ref[...] += p

    @pl.when(c == C_IN - 1)
    def _():
        o_ref[...] = jax.nn.gelu(o_ref[...] + b_ref[0])


def _proj_fused(xmat3, amatT, cwf3, lin_b):
    return pl.pallas_call(
        _proj_fused_body,
        grid=(BN // R_ROWS, C_IN),
        in_specs=[
            pl.BlockSpec((R_ROWS, 1, 1, SEQ_LEN), lambda j, c: (j, c, 0, 0)),
            pl.BlockSpec((E, SEQ_LEN), lambda j, c: (0, 0)),
            pl.BlockSpec((1, E, D_MODEL), lambda j, c: (c, 0, 0)),
            pl.BlockSpec(memory_space=pltpu.SMEM),
        ],
        out_specs=pl.BlockSpec((R_ROWS, D_MODEL), lambda j, c: (j, 0)),
        out_shape=jax.ShapeDtypeStruct((BN, D_MODEL), jnp.float32),
    )(xmat3, amatT, cwf3, lin_b)


def _pearson_body(t_ref, w_ref, u_ref, p_ref, m_ref):
    b = pl.program_id(0)
    t = t_ref[0]
    ct = t - jnp.mean(t, axis=-1, keepdims=True)
    vx = lax.dot_general(ct, w_ref[...], (((1,), (1,)), ((), ())),
                         preferred_element_type=jnp.float32)
    vh = lax.dot_general(ct, u_ref[...], (((1,), (1,)), ((), ())),
                         preferred_element_type=jnp.float32)
    vx = vx / jnp.sqrt(jnp.sum(vx * vx, axis=1, keepdims=True) + EPS)
    vh = vh / jnp.sqrt(jnp.sum(vh * vh, axis=1, keepdims=True) + EPS)
    p = lax.dot_general(vx, vh, (((1,), (1,)), ((), ())),
                        preferred_element_type=jnp.float32)
    p_ref[0] = p

    @pl.when(b == 0)
    def _():
        m_ref[...] = p * 0.25

    @pl.when(b > 0)
    def _():
        m_ref[...] += p * 0.25


def _pearson(target, W_0, U_0):
    return pl.pallas_call(
        _pearson_body,
        grid=(B,),
        in_specs=[
            pl.BlockSpec((1, N, SEQ_LEN), lambda b: (b, 0, 0)),
            pl.BlockSpec((LOW_RANK_D, SEQ_LEN), lambda b: (0, 0)),
            pl.BlockSpec((LOW_RANK_D, SEQ_LEN), lambda b: (0, 0)),
        ],
        out_specs=[
            pl.BlockSpec((1, N, N), lambda b: (b, 0, 0)),
            pl.BlockSpec((N, N), lambda b: (0, 0)),
        ],
        out_shape=[
            jax.ShapeDtypeStruct((B, N, N), jnp.float32),
            jax.ShapeDtypeStruct((N, N), jnp.float32),
        ],
    )(target, W_0, U_0)


def _srt(v):
    k, _ = plsc.sort_key_val(v, v)
    return k


def _rev(v):
    return lax.rev(v, (0,))


def _merge16(a, b):
    rb = _rev(b)
    hi = jnp.maximum(a, rb)
    lo = jnp.minimum(a, rb)
    return _srt(lo), _srt(hi)


def _merge32_top32(a, b):
    a0, a1 = a
    rb0, rb1 = _rev(b[1]), _rev(b[0])
    h0 = jnp.maximum(a0, rb0)
    h1 = jnp.maximum(a1, rb1)
    m0 = jnp.minimum(h0, h1)
    m1 = jnp.maximum(h0, h1)
    return _srt(m0), _srt(m1)


def _row_threshold(v):
    s1 = [_srt(x) for x in v]
    s32 = [_merge16(s1[2 * i], s1[2 * i + 1]) for i in range(4)]
    e32 = _merge32_top32(s32[0], s32[1])
    f32 = _merge32_top32(s32[2], s32[3])
    rb0, rb1 = _rev(f32[1]), _rev(f32[0])
    h0 = jnp.maximum(e32[0], rb0)
    h1 = jnp.maximum(e32[1], rb1)
    return jnp.min(jnp.minimum(h0, h1))


def _topk_softmax_body(pc_hbm, out_hbm, rows_v, out_v):
    wid = lax.axis_index("s") * SC_CORES + lax.axis_index("c")
    base = wid * ROWS_PER_WORKER
    pltpu.sync_copy(pc_hbm.at[pl.ds(base, ROWS_PER_WORKER)], rows_v)

    def row_body(i, carry):
        v = [rows_v[i, pl.ds(16 * j, 16)] for j in range(N_CHUNKS)]
        thr = _row_threshold(v)
        masked = [jnp.where(vj >= thr, vj, 0.0) for vj in v]
        mx = masked[0]
        for mj in masked[1:]:
            mx = jnp.maximum(mx, mj)
        mxs = jnp.max(mx)
        ex = [jnp.exp(mj - mxs) for mj in masked]
        acc = ex[0]
        for ej in ex[1:]:
            acc = acc + ej
        denom = jnp.full((16,), jnp.sum(acc), dtype=jnp.float32)
        for j in range(N_CHUNKS):
            out_v[i, pl.ds(16 * j, 16)] = ex[j] / denom
        return carry

    lax.fori_loop(0, ROWS_PER_WORKER, row_body, 0)
    pltpu.sync_copy(out_v, out_hbm.at[pl.ds(base, ROWS_PER_WORKER)])


@functools.lru_cache(maxsize=1)
def _build_topk_softmax():
    return functools.partial(
        pl.kernel,
        out_type=jax.ShapeDtypeStruct((BN, N), jnp.float32),
        mesh=plsc.VectorSubcoreMesh(
            core_axis_name="c", subcore_axis_name="s",
            num_cores=SC_CORES, num_subcores=SC_SUBCORES),
        scratch_types=[
            pltpu.VMEM((ROWS_PER_WORKER, N), jnp.float32),
            pltpu.VMEM((ROWS_PER_WORKER, N), jnp.float32),
        ],
        compiler_params=pltpu.CompilerParams(needs_layout_passes=False),
    )(_topk_softmax_body)


def kernel(x, target, pearson_sparse_matrix, conv_w, W_0, U_0, lin_w, lin_b):
    del pearson_sparse_matrix
    lw = lin_w[0]
    amatT = jnp.stack(
        [jnp.pad(lw, (TAU * e, SEQ_LEN - L_OUT - TAU * e)) for e in range(E)],
        axis=0)

    pc, pmean = _pearson(target, W_0.T, U_0.T)

    xmat3 = x.transpose(0, 1, 3, 2).reshape(BN, C_IN, 1, SEQ_LEN)
    xmat3, pc = lax.optimization_barrier((xmat3, pc))
    cwf3 = conv_w.transpose(1, 2, 0)
    proj = _proj_fused(xmat3, amatT, cwf3, lin_b)
    mult = _build_topk_softmax()(pc.reshape(BN, N)).reshape(B, N, N)
    return proj, mult, pmean


# speedup vs baseline: 6.3068x; 6.3068x over previous
import functools

import jax
import jax.numpy as jnp
from jax import lax
from jax.experimental import pallas as pl
from jax.experimental.pallas import tpu as pltpu
from jax.experimental.pallas import tpu_sc as plsc

TAU = 8
E = 16
C_IN = 16
D_MODEL = 128
SEQ_LEN = 2048
LOW_RANK_D = 32
EPS = 1e-06
B = 4
N = 128
L_OUT = SEQ_LEN - TAU * (E - 1)
BN = B * N

R_BLK = 8

SC_CORES = 2
SC_SUBCORES = 16
SC_WORKERS = SC_CORES * SC_SUBCORES
ROWS_PER_WORKER = BN // SC_WORKERS
N_CHUNKS = N // 16


M_BLK = 1024


def _conv_fuse_body(x_ref, a_ref, s_ref):
    s_ref[...] = lax.dot_general(x_ref[...], a_ref[...],
                                 (((1,), (1,)), ((), ())),
                                 preferred_element_type=jnp.float32)


def _conv_fuse(xmat, amat):
    m = BN * C_IN
    return pl.pallas_call(
        _conv_fuse_body,
        grid=(m // M_BLK,),
        in_specs=[
            pl.BlockSpec((M_BLK, SEQ_LEN), lambda i: (i, 0)),
            pl.BlockSpec((E, SEQ_LEN), lambda i: (0, 0)),
        ],
        out_specs=pl.BlockSpec((M_BLK, E), lambda i: (i, 0)),
        out_shape=jax.ShapeDtypeStruct((m, E), jnp.float32),
    )(xmat, amat)


def _proj_body(s_ref, w_ref, b_ref, o_ref):
    acc = jnp.dot(s_ref[...], w_ref[...], preferred_element_type=jnp.float32)
    o_ref[...] = jax.nn.gelu(acc + b_ref[0])


def _proj(s2, cwf, lin_b):
    return pl.pallas_call(
        _proj_body,
        in_specs=[
            pl.BlockSpec(memory_space=pltpu.VMEM),
            pl.BlockSpec(memory_space=pltpu.VMEM),
            pl.BlockSpec(memory_space=pltpu.SMEM),
        ],
        out_shape=jax.ShapeDtypeStruct((BN, D_MODEL), jnp.float32),
    )(s2, cwf, lin_b)


def _pearson_body(t_ref, w_ref, u_ref, p_ref, m_ref):
    w = w_ref[...]
    u = u_ref[...]
    acc = None
    for b in range(B):
        t = t_ref[b]
        ct = t - jnp.mean(t, axis=-1, keepdims=True)
        vx = lax.dot_general(ct, w, (((1,), (1,)), ((), ())),
                             preferred_element_type=jnp.float32)
        vh = lax.dot_general(ct, u, (((1,), (1,)), ((), ())),
                             preferred_element_type=jnp.float32)
        vx = vx / jnp.sqrt(jnp.sum(vx * vx, axis=1, keepdims=True) + EPS)
        vh = vh / jnp.sqrt(jnp.sum(vh * vh, axis=1, keepdims=True) + EPS)
        p = lax.dot_general(vx, vh, (((1,), (1,)), ((), ())),
                            preferred_element_type=jnp.float32)
        p_ref[b] = p
        acc = p if acc is None else acc + p
    m_ref[...] = acc * (1.0 / B)


def _pearson(target, W_0, U_0):
    return pl.pallas_call(
        _pearson_body,
        out_shape=[
            jax.ShapeDtypeStruct((B, N, N), jnp.float32),
            jax.ShapeDtypeStruct((N, N), jnp.float32),
        ],
    )(target, W_0, U_0)


def _srt(v):
    k, _ = plsc.sort_key_val(v, v)
    return k


def _rev(v):
    return lax.rev(v, (0,))


def _merge16(a, b):
    rb = _rev(b)
    hi = jnp.maximum(a, rb)
    lo = jnp.minimum(a, rb)
    return _srt(lo), _srt(hi)


def _merge32_top32(a, b):
    a0, a1 = a
    rb0, rb1 = _rev(b[1]), _rev(b[0])
    h0 = jnp.maximum(a0, rb0)
    h1 = jnp.maximum(a1, rb1)
    m0 = jnp.minimum(h0, h1)
    m1 = jnp.maximum(h0, h1)
    return _srt(m0), _srt(m1)


def _row_threshold(v):
    s1 = [_srt(x) for x in v]
    s32 = [_merge16(s1[2 * i], s1[2 * i + 1]) for i in range(4)]
    e32 = _merge32_top32(s32[0], s32[1])
    f32 = _merge32_top32(s32[2], s32[3])
    rb0, rb1 = _rev(f32[1]), _rev(f32[0])
    h0 = jnp.maximum(e32[0], rb0)
    h1 = jnp.maximum(e32[1], rb1)
    return jnp.min(jnp.minimum(h0, h1))


def _topk_softmax_body(pc_hbm, out_hbm, rows_v, out_v):
    wid = lax.axis_index("s") * SC_CORES + lax.axis_index("c")
    base = wid * ROWS_PER_WORKER
    pltpu.sync_copy(pc_hbm.at[pl.ds(base, ROWS_PER_WORKER)], rows_v)

    def row_body(i, carry):
        v = [rows_v[i, pl.ds(16 * j, 16)] for j in range(N_CHUNKS)]
        thr = _row_threshold(v)
        masked = [jnp.where(vj >= thr, vj, 0.0) for vj in v]
        mx = masked[0]
        for mj in masked[1:]:
            mx = jnp.maximum(mx, mj)
        mxs = jnp.max(mx)
        ex = [jnp.exp(mj - mxs) for mj in masked]
        acc = ex[0]
        for ej in ex[1:]:
            acc = acc + ej
        denom = jnp.full((16,), jnp.sum(acc), dtype=jnp.float32)
        for j in range(N_CHUNKS):
            out_v[i, pl.ds(16 * j, 16)] = ex[j] / denom
        return carry

    lax.fori_loop(0, ROWS_PER_WORKER, row_body, 0)
    pltpu.sync_copy(out_v, out_hbm.at[pl.ds(base, ROWS_PER_WORKER)])


@functools.lru_cache(maxsize=1)
def _build_topk_softmax():
    return functools.partial(
        pl.kernel,
        out_type=jax.ShapeDtypeStruct((BN, N), jnp.float32),
        mesh=plsc.VectorSubcoreMesh(
            core_axis_name="c", subcore_axis_name="s",
            num_cores=SC_CORES, num_subcores=SC_SUBCORES),
        scratch_types=[
            pltpu.VMEM((ROWS_PER_WORKER, N), jnp.float32),
            pltpu.VMEM((ROWS_PER_WORKER, N), jnp.float32),
        ],
        compiler_params=pltpu.CompilerParams(needs_layout_passes=False),
    )(_topk_softmax_body)


def kernel(x, target, pearson_sparse_matrix, conv_w, W_0, U_0, lin_w, lin_b):
    del pearson_sparse_matrix
    lw = lin_w[0]
    amat = jnp.stack(
        [jnp.pad(lw, (TAU * e, SEQ_LEN - L_OUT - TAU * e)) for e in range(E)],
        axis=0)

    pc, pmean = _pearson(target, W_0.T, U_0.T)

    xmat = x.transpose(0, 1, 3, 2).reshape(BN * C_IN, SEQ_LEN)
    xmat, pc = lax.optimization_barrier((xmat, pc))
    s = _conv_fuse(xmat, amat)
    s2 = s.reshape(BN, C_IN * E)
    cwf = conv_w.transpose(1, 2, 0).reshape(C_IN * E, D_MODEL)
    proj = _proj(s2, cwf, lin_b)
    mult = _build_topk_softmax()(pc.reshape(BN, N)).reshape(B, N, N)
    return proj, mult, pmean


# speedup vs baseline: 6.4966x; 1.0301x over previous
import functools

import jax
import jax.numpy as jnp
from jax import lax
from jax.experimental import pallas as pl
from jax.experimental.pallas import tpu as pltpu
from jax.experimental.pallas import tpu_sc as plsc

TAU = 8
E = 16
C_IN = 16
D_MODEL = 128
SEQ_LEN = 2048
LOW_RANK_D = 32
EPS = 1e-06
B = 4
N = 128
L_OUT = SEQ_LEN - TAU * (E - 1)
BN = B * N

R_BLK = 8

SC_CORES = 1
SC_SUBCORES = 16
SC_WORKERS = SC_CORES * SC_SUBCORES
ROWS_PER_WORKER = BN // SC_WORKERS
N_CHUNKS = N // 16


M_BLK = 1024


def _conv_fuse_body(x_ref, a_ref, s_ref):
    s_ref[...] = lax.dot_general(x_ref[...], a_ref[...],
                                 (((1,), (1,)), ((), ())),
                                 preferred_element_type=jnp.float32)


def _conv_fuse(xmat, amat):
    m = BN * C_IN
    return pl.pallas_call(
        _conv_fuse_body,
        grid=(m // M_BLK,),
        in_specs=[
            pl.BlockSpec((M_BLK, SEQ_LEN), lambda i: (i, 0)),
            pl.BlockSpec((E, SEQ_LEN), lambda i: (0, 0)),
        ],
        out_specs=pl.BlockSpec((M_BLK, E), lambda i: (i, 0)),
        out_shape=jax.ShapeDtypeStruct((m, E), jnp.float32),
    )(xmat, amat)


def _proj_body(s_ref, w_ref, b_ref, o_ref):
    acc = jnp.dot(s_ref[...], w_ref[...], preferred_element_type=jnp.float32)
    o_ref[...] = jax.nn.gelu(acc + b_ref[0])


def _proj(s2, cwf, lin_b):
    return pl.pallas_call(
        _proj_body,
        in_specs=[
            pl.BlockSpec(memory_space=pltpu.VMEM),
            pl.BlockSpec(memory_space=pltpu.VMEM),
            pl.BlockSpec(memory_space=pltpu.SMEM),
        ],
        out_shape=jax.ShapeDtypeStruct((BN, D_MODEL), jnp.float32),
    )(s2, cwf, lin_b)


def _pearson_body(t_ref, w_ref, u_ref, p_ref, m_ref):
    w = w_ref[...]
    u = u_ref[...]
    acc = None
    for b in range(B):
        t = t_ref[b]
        ct = t - jnp.mean(t, axis=-1, keepdims=True)
        vx = lax.dot_general(ct, w, (((1,), (1,)), ((), ())),
                             preferred_element_type=jnp.float32)
        vh = lax.dot_general(ct, u, (((1,), (1,)), ((), ())),
                             preferred_element_type=jnp.float32)
        vx = vx / jnp.sqrt(jnp.sum(vx * vx, axis=1, keepdims=True) + EPS)
        vh = vh / jnp.sqrt(jnp.sum(vh * vh, axis=1, keepdims=True) + EPS)
        p = lax.dot_general(vx, vh, (((1,), (1,)), ((), ())),
                            preferred_element_type=jnp.float32)
        p_ref[b] = p
        acc = p if acc is None else acc + p
    m_ref[...] = acc * (1.0 / B)


def _pearson(target, W_0, U_0):
    return pl.pallas_call(
        _pearson_body,
        out_shape=[
            jax.ShapeDtypeStruct((B, N, N), jnp.float32),
            jax.ShapeDtypeStruct((N, N), jnp.float32),
        ],
    )(target, W_0, U_0)


def _srt(v):
    k, _ = plsc.sort_key_val(v, v)
    return k


def _rev(v):
    return lax.rev(v, (0,))


def _merge16(a, b):
    rb = _rev(b)
    hi = jnp.maximum(a, rb)
    lo = jnp.minimum(a, rb)
    return _srt(lo), _srt(hi)


def _merge32_top32(a, b):
    a0, a1 = a
    rb0, rb1 = _rev(b[1]), _rev(b[0])
    h0 = jnp.maximum(a0, rb0)
    h1 = jnp.maximum(a1, rb1)
    m0 = jnp.minimum(h0, h1)
    m1 = jnp.maximum(h0, h1)
    return _srt(m0), _srt(m1)


def _row_threshold(v):
    s1 = [_srt(x) for x in v]
    s32 = [_merge16(s1[2 * i], s1[2 * i + 1]) for i in range(4)]
    e32 = _merge32_top32(s32[0], s32[1])
    f32 = _merge32_top32(s32[2], s32[3])
    rb0, rb1 = _rev(f32[1]), _rev(f32[0])
    h0 = jnp.maximum(e32[0], rb0)
    h1 = jnp.maximum(e32[1], rb1)
    return jnp.min(jnp.minimum(h0, h1))


def _topk_softmax_body(pc_hbm, out_hbm, rows_v, out_v):
    wid = lax.axis_index("s") * SC_CORES + lax.axis_index("c")
    base = wid * ROWS_PER_WORKER
    pltpu.sync_copy(pc_hbm.at[pl.ds(base, ROWS_PER_WORKER)], rows_v)

    def row_body(i, carry):
        v = [rows_v[i, pl.ds(16 * j, 16)] for j in range(N_CHUNKS)]
        thr = _row_threshold(v)
        masked = [jnp.where(vj >= thr, vj, 0.0) for vj in v]
        mx = masked[0]
        for mj in masked[1:]:
            mx = jnp.maximum(mx, mj)
        mxs = jnp.max(mx)
        ex = [jnp.exp(mj - mxs) for mj in masked]
        acc = ex[0]
        for ej in ex[1:]:
            acc = acc + ej
        denom = jnp.full((16,), jnp.sum(acc), dtype=jnp.float32)
        for j in range(N_CHUNKS):
            out_v[i, pl.ds(16 * j, 16)] = ex[j] / denom
        return carry

    lax.fori_loop(0, ROWS_PER_WORKER, row_body, 0)
    pltpu.sync_copy(out_v, out_hbm.at[pl.ds(base, ROWS_PER_WORKER)])


@functools.lru_cache(maxsize=1)
def _build_topk_softmax():
    return functools.partial(
        pl.kernel,
        out_type=jax.ShapeDtypeStruct((BN, N), jnp.float32),
        mesh=plsc.VectorSubcoreMesh(
            core_axis_name="c", subcore_axis_name="s",
            num_cores=SC_CORES, num_subcores=SC_SUBCORES),
        scratch_types=[
            pltpu.VMEM((ROWS_PER_WORKER, N), jnp.float32),
            pltpu.VMEM((ROWS_PER_WORKER, N), jnp.float32),
        ],
        compiler_params=pltpu.CompilerParams(needs_layout_passes=False),
    )(_topk_softmax_body)


def kernel(x, target, pearson_sparse_matrix, conv_w, W_0, U_0, lin_w, lin_b):
    del pearson_sparse_matrix
    lw = lin_w[0]
    amat = jnp.stack(
        [jnp.pad(lw, (TAU * e, SEQ_LEN - L_OUT - TAU * e)) for e in range(E)],
        axis=0)

    pc, pmean = _pearson(target, W_0.T, U_0.T)

    xmat = x.transpose(0, 1, 3, 2).reshape(BN * C_IN, SEQ_LEN)
    xmat, pc = lax.optimization_barrier((xmat, pc))
    s = _conv_fuse(xmat, amat)
    s2 = s.reshape(BN, C_IN * E)
    cwf = conv_w.transpose(1, 2, 0).reshape(C_IN * E, D_MODEL)
    proj = _proj(s2, cwf, lin_b)
    mult = _build_topk_softmax()(pc.reshape(BN, N)).reshape(B, N, N)
    return proj, mult, pmean
